# baseline (speedup 1.0000x reference)
import functools

import jax
import jax.numpy as jnp
from jax import lax
from jax.experimental import pallas as pl
from jax.experimental.pallas import tpu as pltpu
from jax.experimental.pallas import tpu_sc as plsc

N, C, H, W = 4, 96, 224, 224
SPATIAL_SCALE_CONST = 1.0

NWORKERS = 32
K = 128
TW = 128
HB = 16
NROW = H // HB


def _pack_words(t):
    lo = t[:, :C // 2].astype(jnp.bfloat16)
    hi = t[:, C // 2:].astype(jnp.bfloat16)
    lo_u = jax.lax.bitcast_convert_type(lo, jnp.uint16).astype(jnp.int32)
    hi_u = jax.lax.bitcast_convert_type(hi, jnp.uint16).astype(jnp.int32)
    return lo_u | (hi_u << 16)


def _transpose_body(f_ref, fn_ref, o_ref):
    hbw = HB * W
    blk = f_ref[0].reshape(C, hbw).T
    nxt = fn_ref[0, :, 0:1, 0:1].reshape(C, 1).T
    wa = _pack_words(blk)
    wb = jnp.concatenate([wa[1:], _pack_words(nxt)], axis=0)
    zpad = jnp.zeros((hbw, 16), jnp.int32)
    o_ref[...] = jnp.concatenate([wa, zpad, wb, zpad], axis=1)


def _build_table(features):
    grid = (N, NROW)

    def nxt_map(n, h):
        at_end = h == NROW - 1
        last = at_end & (n == N - 1)
        nn = jnp.where(at_end & jnp.logical_not(last), n + 1, n)
        nh = jnp.where(last, h, jnp.where(at_end, 0, h + 1))
        return (nn, 0, nh, 0)

    return pl.pallas_call(
        _transpose_body,
        grid=grid,
        in_specs=[
            pl.BlockSpec((1, C, HB, W), lambda n, h: (n, 0, h, 0)),
            pl.BlockSpec((1, C, HB, W), nxt_map),
        ],
        out_specs=pl.BlockSpec((HB * W, TW), lambda n, h: (n * NROW + h, 0)),
        out_shape=jax.ShapeDtypeStruct((N * H * W, TW), jnp.int32),
    )(features, features)


def _make_sc_interp(r, r_pad):
    pb = r_pad // NWORKERS
    nc = pb // K
    assert nc % 2 == 0 and nc >= 4
    mesh = plsc.VectorSubcoreMesh(core_axis_name="c", subcore_axis_name="s")

    scratch = [pltpu.VMEM((pb,), jnp.float32) for _ in range(3)]
    scratch += [pltpu.VMEM((K,), jnp.int32) for _ in range(4)]
    scratch += [pltpu.VMEM((K,), jnp.float32) for _ in range(8)]
    scratch += [pltpu.VMEM((K, TW), jnp.int32) for _ in range(4)]
    scratch += [pltpu.VMEM((K, C), jnp.float32) for _ in range(2)]
    scratch += [pltpu.SemaphoreType.DMA for _ in range(4)]

    @functools.partial(
        pl.kernel,
        mesh=mesh,
        out_type=jax.ShapeDtypeStruct((r, C), jnp.float32),
        scratch_types=scratch,
    )
    def sc_interp(bs_hbm, xs_hbm, ys_hbm, table_hbm, out_hbm,
                  bs_v, xs_v, ys_v, *rest):
        idx = [rest[0:2], rest[2:4]]
        wts = [rest[4:8], rest[8:12]]
        rows = [rest[12:14], rest[14:16]]
        outv = [rest[16], rest[17]]
        gsem = [rest[18], rest[19]]
        osem = [rest[20], rest[21]]

        wid = lax.axis_index("s") * 2 + lax.axis_index("c")
        wbase = wid * pb

        pltpu.sync_copy(bs_hbm.at[pl.ds(wbase, pb)], bs_v)
        pltpu.sync_copy(xs_hbm.at[pl.ds(wbase, pb)], xs_v)
        pltpu.sync_copy(ys_hbm.at[pl.ds(wbase, pb)], ys_v)

        def stage(s, off):
            for j in range(K // 16):
                sl = pl.ds(off + j * 16, 16)
                b = bs_v[sl].astype(jnp.int32)
                x = jnp.minimum(jnp.maximum(xs_v[sl] * SPATIAL_SCALE_CONST,
                                            0.0), float(W - 1))
                y = jnp.minimum(jnp.maximum(ys_v[sl] * SPATIAL_SCALE_CONST,
                                            0.0), float(H - 1))
                x0 = x.astype(jnp.int32)
                y0 = y.astype(jnp.int32)
                lx = x - x0.astype(jnp.float32)
                ly = y - y0.astype(jnp.float32)
                dy = jnp.where(y0 < H - 1, W, 0)
                ib = (b * H + y0) * W + x0
                sj = pl.ds(j * 16, 16)
                idx[s][0][sj] = ib
                idx[s][1][sj] = ib + dy
                hx = 1.0 - lx
                hy = 1.0 - ly
                wts[s][0][sj] = hy * hx
                wts[s][1][sj] = hy * lx
                wts[s][2][sj] = ly * hx
                wts[s][3][sj] = ly * lx
            for c in range(2):
                pltpu.async_copy(table_hbm.at[idx[s][c]], rows[s][c], gsem[s])

        def process(s, base, t):
            for c in range(2):
                pltpu.make_async_copy(
                    table_hbm.at[idx[s][c]], rows[s][c], gsem[s]).wait()

            prev_valid = (t >= 2) & (base - 2 * K + K <= r)

            @pl.when(prev_valid)
            def _():
                pltpu.make_async_copy(
                    outv[s], out_hbm.at[pl.ds(0, K)], osem[s]).wait()

            ry0, ry1 = rows[s]
            w0, w1, w2, w3 = wts[s]
            ov = outv[s]

            def point_group(q, carry2):
                qb = q * 16
                wv0 = w0[pl.ds(qb, 16)]
                wv1 = w1[pl.ds(qb, 16)]
                wv2 = w2[pl.ds(qb, 16)]
                wv3 = w3[pl.ds(qb, 16)]
                hi_mask = jnp.full((16,), -65536, jnp.int32)
                for lane in range(16):
                    p = qb + lane
                    a0 = wv0[lane]
                    a1 = wv1[lane]
                    a2 = wv2[lane]
                    a3 = wv3[lane]
                    for g in range(C // 32):
                        w00 = ry0[p, pl.ds(g * 16, 16)]
                        w01 = ry0[p, pl.ds(64 + g * 16, 16)]
                        w10 = ry1[p, pl.ds(g * 16, 16)]
                        w11 = ry1[p, pl.ds(64 + g * 16, 16)]
                        acc_e = (
                            a0 * lax.bitcast_convert_type(w00 << 16, jnp.float32)
                            + a1 * lax.bitcast_convert_type(w01 << 16, jnp.float32)
                            + a2 * lax.bitcast_convert_type(w10 << 16, jnp.float32)
                            + a3 * lax.bitcast_convert_type(w11 << 16, jnp.float32))
                        acc_o = (
                            a0 * lax.bitcast_convert_type(w00 & hi_mask, jnp.float32)
                            + a1 * lax.bitcast_convert_type(w01 & hi_mask, jnp.float32)
                            + a2 * lax.bitcast_convert_type(w10 & hi_mask, jnp.float32)
                            + a3 * lax.bitcast_convert_type(w11 & hi_mask, jnp.float32))
                        ov[p, pl.ds(g * 16, 16)] = acc_e
                        ov[p, pl.ds(48 + g * 16, 16)] = acc_o
                return carry2

            lax.fori_loop(0, K // 16, point_group, 0)

            @pl.when(base + K <= r)
            def _():
                pltpu.async_copy(ov, out_hbm.at[pl.ds(base, K)], osem[s])

            rem = r % K
            if rem:
                @pl.when((base < r) & (base + K > r))
                def _():
                    pltpu.sync_copy(ov.at[pl.ds(0, rem)],
                                    out_hbm.at[pl.ds(base, rem)])


        stage(0, 0)
        stage(1, K)

        def pair(p2, carry):
            t0 = 2 * p2
            off0 = t0 * K
            b0 = wbase + off0
            process(0, b0, t0)

            @pl.when(t0 + 2 < nc)
            def _():
                stage(0, off0 + 2 * K)

            t1 = t0 + 1
            b1 = b0 + K
            process(1, b1, t1)

            @pl.when(t1 + 2 < nc)
            def _():
                stage(1, off0 + 3 * K)

            return carry

        lax.fori_loop(0, nc // 2, pair, 0)

        for s in range(2):
            @pl.when(wbase + (nc - 2 + s) * K + K <= r)
            def _():
                pltpu.make_async_copy(
                    outv[s], out_hbm.at[pl.ds(0, K)], osem[s]).wait()

    return sc_interp


def kernel(features, rois):
    r = rois.shape[0]
    chunk_stride = NWORKERS * K * 2
    r_pad = ((r + chunk_stride - 1) // chunk_stride) * chunk_stride

    table = _build_table(features)

    bs = rois[:, 0]
    xs = rois[:, 1]
    ys = rois[:, 2]
    pad = r_pad - r
    if pad:
        z = jnp.zeros((pad,), jnp.float32)
        bs = jnp.concatenate([bs, z])
        xs = jnp.concatenate([xs, z])
        ys = jnp.concatenate([ys, z])

    return _make_sc_interp(r, r_pad)(bs, xs, ys, table)


# speedup vs baseline: 1.9644x; 1.9644x over previous
import functools

import jax
import jax.numpy as jnp
from jax import lax
from jax.experimental import pallas as pl
from jax.experimental.pallas import tpu as pltpu
from jax.experimental.pallas import tpu_sc as plsc

N, C, H, W = 4, 96, 224, 224
SPATIAL_SCALE_CONST = 1.0

NWORKERS = 32
K = 128
TW = 128
HB = 16
NROW = H // HB


def _pack_words(t):
    lo = t[:, :C // 2].astype(jnp.bfloat16)
    hi = t[:, C // 2:].astype(jnp.bfloat16)
    lo_u = jax.lax.bitcast_convert_type(lo, jnp.uint16).astype(jnp.int32)
    hi_u = jax.lax.bitcast_convert_type(hi, jnp.uint16).astype(jnp.int32)
    return lo_u | (hi_u << 16)


def _transpose_body(f_ref, fn_ref, o_ref):
    hbw = HB * W
    blk = f_ref[0].reshape(C, hbw).T
    nxt = fn_ref[0, :, 0:1, 0:1].reshape(C, 1).T
    wa = _pack_words(blk)
    wb = jnp.concatenate([wa[1:], _pack_words(nxt)], axis=0)
    zpad = jnp.zeros((hbw, 16), jnp.int32)
    o_ref[...] = jnp.concatenate([wa, zpad, wb, zpad], axis=1)


def _build_table(features):
    grid = (N, NROW)

    def nxt_map(n, h):
        at_end = h == NROW - 1
        last = at_end & (n == N - 1)
        nn = jnp.where(at_end & jnp.logical_not(last), n + 1, n)
        nh = jnp.where(last, h, jnp.where(at_end, 0, h + 1))
        return (nn, 0, nh, 0)

    return pl.pallas_call(
        _transpose_body,
        grid=grid,
        in_specs=[
            pl.BlockSpec((1, C, HB, W), lambda n, h: (n, 0, h, 0)),
            pl.BlockSpec((1, C, HB, W), nxt_map),
        ],
        out_specs=pl.BlockSpec((HB * W, TW), lambda n, h: (n * NROW + h, 0)),
        out_shape=jax.ShapeDtypeStruct((N * H * W, TW), jnp.int32),
    )(features, features)


def _make_sc_interp(r, r_pad):
    pb = r_pad // NWORKERS
    nc = pb // K
    assert nc % 2 == 0 and nc >= 4
    mesh = plsc.VectorSubcoreMesh(core_axis_name="c", subcore_axis_name="s")

    scratch = [pltpu.VMEM((pb,), jnp.float32) for _ in range(3)]
    scratch += [pltpu.VMEM((K,), jnp.int32) for _ in range(4)]
    scratch += [pltpu.VMEM((K,), jnp.float32) for _ in range(8)]
    scratch += [pltpu.VMEM((K, TW), jnp.int32) for _ in range(4)]
    scratch += [pltpu.VMEM((K, C), jnp.float32) for _ in range(2)]
    scratch += [pltpu.SemaphoreType.DMA for _ in range(4)]

    @functools.partial(
        pl.kernel,
        mesh=mesh,
        out_type=jax.ShapeDtypeStruct((r, C), jnp.float32),
        scratch_types=scratch,
    )
    def sc_interp(bs_hbm, xs_hbm, ys_hbm, table_hbm, out_hbm,
                  bs_v, xs_v, ys_v, *rest):
        idx = [rest[0:2], rest[2:4]]
        wts = [rest[4:8], rest[8:12]]
        rows = [rest[12:14], rest[14:16]]
        outv = [rest[16], rest[17]]
        gsem = [rest[18], rest[19]]
        osem = [rest[20], rest[21]]

        wid = lax.axis_index("s") * 2 + lax.axis_index("c")
        wbase = wid * pb

        pltpu.sync_copy(bs_hbm.at[pl.ds(wbase, pb)], bs_v)
        pltpu.sync_copy(xs_hbm.at[pl.ds(wbase, pb)], xs_v)
        pltpu.sync_copy(ys_hbm.at[pl.ds(wbase, pb)], ys_v)

        def stage(s, off):
            for j in range(K // 16):
                sl = pl.ds(off + j * 16, 16)
                b = bs_v[sl].astype(jnp.int32)
                x = jnp.minimum(jnp.maximum(xs_v[sl] * SPATIAL_SCALE_CONST,
                                            0.0), float(W - 1))
                y = jnp.minimum(jnp.maximum(ys_v[sl] * SPATIAL_SCALE_CONST,
                                            0.0), float(H - 1))
                x0 = x.astype(jnp.int32)
                y0 = y.astype(jnp.int32)
                lx = x - x0.astype(jnp.float32)
                ly = y - y0.astype(jnp.float32)
                dy = jnp.where(y0 < H - 1, W, 0)
                ib = (b * H + y0) * W + x0
                sj = pl.ds(j * 16, 16)
                idx[s][0][sj] = ib
                idx[s][1][sj] = ib + dy
                hx = 1.0 - lx
                hy = 1.0 - ly
                wts[s][0][sj] = hy * hx
                wts[s][1][sj] = hy * lx
                wts[s][2][sj] = ly * hx
                wts[s][3][sj] = ly * lx
            for c in range(2):
                pltpu.async_copy(table_hbm.at[idx[s][c]], rows[s][c], gsem[s])

        def process(s, base, t):
            for c in range(2):
                pltpu.make_async_copy(
                    table_hbm.at[idx[s][c]], rows[s][c], gsem[s]).wait()

            prev_valid = (t >= 2) & (base - 2 * K + K <= r)

            @pl.when(prev_valid)
            def _():
                pltpu.make_async_copy(
                    outv[s], out_hbm.at[pl.ds(0, K)], osem[s]).wait()

            ry0, ry1 = rows[s]
            w0, w1, w2, w3 = wts[s]
            ov = outv[s]

            def point_group(q, carry2):
                qb = q * 16
                wv0 = w0[pl.ds(qb, 16)]
                wv1 = w1[pl.ds(qb, 16)]
                wv2 = w2[pl.ds(qb, 16)]
                wv3 = w3[pl.ds(qb, 16)]
                hi_mask = jnp.full((16,), -65536, jnp.int32)
                for lane in range(16):
                    p = qb + lane
                    a0 = wv0[lane]
                    a1 = wv1[lane]
                    a2 = wv2[lane]
                    a3 = wv3[lane]
                    for g in range(C // 32):
                        w00 = ry0[p, pl.ds(g * 16, 16)]
                        w01 = ry0[p, pl.ds(64 + g * 16, 16)]
                        w10 = ry1[p, pl.ds(g * 16, 16)]
                        w11 = ry1[p, pl.ds(64 + g * 16, 16)]
                        acc_e = (
                            a0 * lax.bitcast_convert_type(w00 << 16, jnp.float32)
                            + a1 * lax.bitcast_convert_type(w01 << 16, jnp.float32)
                            + a2 * lax.bitcast_convert_type(w10 << 16, jnp.float32)
                            + a3 * lax.bitcast_convert_type(w11 << 16, jnp.float32))
                        acc_o = (
                            a0 * lax.bitcast_convert_type(w00 & hi_mask, jnp.float32)
                            + a1 * lax.bitcast_convert_type(w01 & hi_mask, jnp.float32)
                            + a2 * lax.bitcast_convert_type(w10 & hi_mask, jnp.float32)
                            + a3 * lax.bitcast_convert_type(w11 & hi_mask, jnp.float32))
                        ov[p, pl.ds(g * 16, 16)] = acc_e
                        ov[p, pl.ds(48 + g * 16, 16)] = acc_o
                return carry2

            lax.fori_loop(0, K // 16, point_group, 0)

            @pl.when(base + K <= r)
            def _():
                pltpu.async_copy(ov, out_hbm.at[pl.ds(base, K)], osem[s])

            rem = r % K
            if rem:
                @pl.when((base < r) & (base + K > r))
                def _():
                    pltpu.sync_copy(ov.at[pl.ds(0, rem)],
                                    out_hbm.at[pl.ds(base, rem)])


        stage(0, 0)
        stage(1, K)

        def pair(p2, carry):
            t0 = 2 * p2
            off0 = t0 * K
            b0 = wbase + off0
            process(0, b0, t0)

            @pl.when(t0 + 2 < nc)
            def _():
                stage(0, off0 + 2 * K)

            t1 = t0 + 1
            b1 = b0 + K
            process(1, b1, t1)

            @pl.when(t1 + 2 < nc)
            def _():
                stage(1, off0 + 3 * K)

            return carry

        lax.fori_loop(0, nc // 2, pair, 0)

        for s in range(2):
            @pl.when(wbase + (nc - 2 + s) * K + K <= r)
            def _():
                pltpu.make_async_copy(
                    outv[s], out_hbm.at[pl.ds(0, K)], osem[s]).wait()

    return sc_interp


def kernel(features, rois):
    r = rois.shape[0]
    chunk_stride = NWORKERS * K * 2
    r_pad = ((r + chunk_stride - 1) // chunk_stride) * chunk_stride

    table = _build_table(features)

    bs = rois[:, 0]
    xs = rois[:, 1]
    ys = rois[:, 2]
    pad = r_pad - r
    if pad:
        bs = jnp.concatenate([bs, bs[:pad]])
        xs = jnp.concatenate([xs, xs[:pad]])
        ys = jnp.concatenate([ys, ys[:pad]])

    return _make_sc_interp(r, r_pad)(bs, xs, ys, table)
